# baseline (speedup 1.0000x reference)
import numpy as np
import jax
import jax.numpy as jnp
from jax import lax
from jax.experimental import pallas as pl
from jax.experimental.pallas import tpu as pltpu

_POOL = 16
_SEQ = 128
_EMBED = 512
_HEADS = 8
_HEAD_DIM = 64
_GROUP = 2


def _fold_bn(gamma, beta, mean, var, eps=1e-5):
    s = gamma / jnp.sqrt(var + eps)
    return s, beta - mean * s


def _adaptive_pool_matrix(n_in, n_out=_POOL):
    m = np.zeros((n_out, n_in), np.float32)
    for i in range(n_out):
        s = (i * n_in) // n_out
        e = -(-((i + 1) * n_in) // n_out)
        m[i, s:e] = 1.0 / (e - s)
    return m


def _pool2d_matrix_t(h_in, w_in):
    return np.kron(_adaptive_pool_matrix(h_in), _adaptive_pool_matrix(w_in)).T.copy()


def _pool_big_matrix_t(gh, gw, stride, row_pitch, n_rows):
    ph_m = _adaptive_pool_matrix(gh)
    pw_m = _adaptive_pool_matrix(gw)
    m = np.zeros((n_rows, _POOL * _POOL), np.float32)
    for ph in range(gh):
        for pw in range(gw):
            q = ph * row_pitch + stride * pw
            m[q, :] = np.kron(ph_m[:, ph], pw_m[:, pw])
    return m


def _gelu_tanh(x):
    c = 0.7978845608028654
    return 0.5 * x * (1.0 + jnp.tanh(c * (x + 0.044715 * x * x * x)))


def _layernorm(x, g, b, eps=1e-5):
    m = jnp.mean(x, axis=-1, keepdims=True)
    c = x - m
    v = jnp.mean(c * c, axis=-1, keepdims=True)
    return c * lax.rsqrt(v + eps) * g + b


def _const(arr):
    zeros = (0,) * arr.ndim
    return pl.BlockSpec(arr.shape, lambda n: zeros)


def _mmT(a, b):
    return lax.dot_general(a, b, (((0,), (0,)), ((), ())),
                           preferred_element_type=jnp.float32)


def _branches(H, W, y, w2_ref, b2_ref, w3_ref, b3_ref, p2_ref, p3_ref):
    C = 32
    h3 = (H - 3) // 3 + 1
    h5 = (H - 5) // 5 + 1
    y3 = y.reshape(H, W, C)

    n2 = h3 * W - 2
    y2g = y3[:3 * h3].reshape(h3, 3, W, C)
    a2 = jnp.concatenate(
        [lax.slice(y2g[:, di].reshape(h3 * W, C), (dj, 0), (dj + n2, C))
         for di in range(3) for dj in range(3)], axis=1)
    z2 = jnp.maximum(b2_ref[...] + jnp.dot(a2, w2_ref[...],
                                           preferred_element_type=jnp.float32),
                     0.0).astype(jnp.bfloat16)
    c2 = _mmT(z2, p2_ref[...])

    n3 = h5 * W - 4
    y3g = y3[:5 * h5].reshape(h5, 5, W, C)
    a3 = jnp.concatenate(
        [lax.slice(y3g[:, di].reshape(h5 * W, C), (dj, 0), (dj + n3, C))
         for di in range(5) for dj in range(5)], axis=1)
    z3 = jnp.maximum(b3_ref[...] + jnp.dot(a3, w3_ref[...],
                                           preferred_element_type=jnp.float32),
                     0.0).astype(jnp.bfloat16)
    c3 = _mmT(z3, p3_ref[...])
    return c2, c3


def _fused_body(H, W, x_ref, w0_ref, b0_ref, w1_ref, b1_ref,
                w2_ref, b2_ref, w3_ref, b3_ref,
                p1_ref, p2_ref, p3_ref,
                wfc_ref, bfc_ref, wqkv_ref, bqkv_ref,
                wm_ref, bm_ref,
                g1_ref, be1_ref, g2_ref, be2_ref,
                wf1_ref, bf1_ref, wf2_ref, bf2_ref, c4_ref, out_ref):
    G = _GROUP
    HW = H * W
    D = _EMBED

    bf = jnp.bfloat16
    ys = [jnp.maximum(_mmT(x_ref[g].astype(bf), w0_ref[...]) + b0_ref[...],
                      0.0).astype(bf)
          for g in range(G)]
    y_all = jnp.concatenate(ys, axis=0)
    z1_all = jnp.maximum(jnp.dot(y_all, w1_ref[...],
                                 preferred_element_type=jnp.float32)
                         + b1_ref[...], 0.0).astype(bf)

    vls = []
    for g in range(G):
        c1 = _mmT(lax.slice(z1_all, (g * HW, 0), ((g + 1) * HW, 32)),
                  p1_ref[...])
        c2, c3 = _branches(H, W, ys[g], w2_ref, b2_ref, w3_ref, b3_ref,
                           p2_ref, p3_ref)
        vls.append(jnp.concatenate([c1, c2, c3], axis=0))
    vl = jnp.concatenate(vls, axis=0).astype(bf)

    xfc = jnp.maximum(jnp.dot(vl, wfc_ref[...],
                              preferred_element_type=jnp.float32) + bfc_ref[...], 0.0)

    xfc_b = xfc.astype(bf)
    qkv = (jnp.dot(xfc_b, wqkv_ref[...],
                   preferred_element_type=jnp.float32) + bqkv_ref[...]).astype(bf)
    scale = 1.0 / (_HEAD_DIM ** 0.5)
    s_list = []
    for g in range(G):
        r0 = g * _SEQ
        for h in range(_HEADS):
            c0 = h * _HEAD_DIM
            kh = lax.slice(qkv, (r0, D + c0), (r0 + _SEQ, D + c0 + _HEAD_DIM))
            vh = lax.slice(qkv, (r0, 2 * D + c0), (r0 + _SEQ, 2 * D + c0 + _HEAD_DIM))
            s_list.append(lax.dot_general(kh, vh, (((1,), (1,)), ((), ())),
                                          preferred_element_type=jnp.float32))
    s_all = jnp.concatenate(s_list, axis=0) * scale
    s_all = s_all - jnp.max(s_all, axis=-1, keepdims=True)
    p_all = jnp.exp(s_all)
    pb = (p_all / jnp.sum(p_all, axis=-1, keepdims=True)).astype(bf)
    o_rows = []
    for g in range(G):
        r0 = g * _SEQ
        p_cat = jnp.concatenate(
            [lax.slice(pb, ((g * _HEADS + h) * _SEQ, 0),
                       ((g * _HEADS + h) * _SEQ + _SEQ, _SEQ))
             for h in range(_HEADS)], axis=1)
        q_bd = jnp.concatenate(
            [jnp.pad(lax.slice(qkv, (r0, h * _HEAD_DIM),
                               (r0 + _SEQ, (h + 1) * _HEAD_DIM)),
                     ((0, 0), (h * _HEAD_DIM, D - (h + 1) * _HEAD_DIM)))
             for h in range(_HEADS)], axis=0)
        o_rows.append(jnp.dot(p_cat, q_bd, preferred_element_type=jnp.float32))
    o = jnp.concatenate(o_rows, axis=0).astype(bf)
    merged = jnp.dot(o, wm_ref[...], preferred_element_type=jnp.float32) + bm_ref[...]

    h1 = _layernorm(xfc + merged, g1_ref[...], be1_ref[...])

    f = _gelu_tanh((jnp.dot(h1.astype(bf), wf1_ref[...],
                            preferred_element_type=jnp.float32)
                    + bf1_ref[...]).astype(bf))
    f = jnp.dot(f, wf2_ref[...],
                preferred_element_type=jnp.float32) + bf2_ref[...]
    h2 = _layernorm(h1 + f, g2_ref[...], be2_ref[...])

    out_ref[...] = jnp.dot(c4_ref[...], h2.astype(bf),
                           preferred_element_type=jnp.float32).reshape(G, 1, D)


def kernel(x, w0, bn0_g, bn0_b, bn0_m, bn0_v, w1, bn1_g, bn1_b, bn1_m, bn1_v,
           w2, bn2_g, bn2_b, bn2_m, bn2_v, w3, bn3_g, bn3_b, bn3_m, bn3_v,
           wfc, bfc, wq, bq, wk, bk, wv, bv, wm, bm,
           ln1_g, ln1_b, ln2_g, ln2_b, wf1, bf1, wf2, bf2, fmp):
    bs, cin, H, W = x.shape
    h3, w3s = (H - 3) // 3 + 1, (W - 3) // 3 + 1
    h5, w5s = (H - 5) // 5 + 1, (W - 5) // 5 + 1
    G = _GROUP
    assert bs % G == 0

    xr = x.reshape(bs, cin, H * W)

    s0, t0 = _fold_bn(bn0_g, bn0_b, bn0_m, bn0_v)
    w0f, b0 = (w0 * s0[None, :]).astype(jnp.bfloat16), t0.reshape(1, -1)
    s1, t1 = _fold_bn(bn1_g, bn1_b, bn1_m, bn1_v)
    w1f, b1 = (w1 * s1[None, :]).astype(jnp.bfloat16), t1.reshape(1, -1)
    s2, t2 = _fold_bn(bn2_g, bn2_b, bn2_m, bn2_v)
    w2f, b2 = (w2 * s2).reshape(9 * 32, 32).astype(jnp.bfloat16), t2.reshape(1, -1)
    s3, t3 = _fold_bn(bn3_g, bn3_b, bn3_m, bn3_v)
    w3f, b3 = (w3 * s3).reshape(25 * 32, 64).astype(jnp.bfloat16), t3.reshape(1, -1)

    p1t = jnp.asarray(_pool2d_matrix_t(H, W)).astype(jnp.bfloat16)
    p2t = jnp.asarray(_pool_big_matrix_t(h3, w3s, 3, W, h3 * W - 2)).astype(jnp.bfloat16)
    p3t = jnp.asarray(_pool_big_matrix_t(h5, w5s, 5, W, h5 * W - 4)).astype(jnp.bfloat16)

    bfc2 = bfc.reshape(1, _EMBED)
    wfc_b = wfc.astype(jnp.bfloat16)
    wm_b = wm.astype(jnp.bfloat16)
    wf1_b = wf1.astype(jnp.bfloat16)
    wf2_b = wf2.astype(jnp.bfloat16)
    wqkv = jnp.concatenate([wq, wk, wv], axis=1).astype(jnp.bfloat16)
    bqkv = jnp.concatenate([bq, bk, bv]).reshape(1, 3 * _EMBED)
    bm2 = bm.reshape(1, _EMBED)
    g1, be1 = ln1_g.reshape(1, _EMBED), ln1_b.reshape(1, _EMBED)
    g2, be2 = ln2_g.reshape(1, _EMBED), ln2_b.reshape(1, _EMBED)
    bf1_2 = bf1.reshape(1, 2 * _EMBED)
    bf2_2 = bf2.reshape(1, _EMBED)
    cvec = jnp.mean(fmp[0], axis=1).reshape(1, _SEQ)
    c4 = jnp.kron(jnp.eye(G, dtype=jnp.float32), cvec).astype(jnp.bfloat16)

    def body(*refs):
        _fused_body(H, W, *refs)

    out = pl.pallas_call(
        body,
        out_shape=jax.ShapeDtypeStruct((bs, 1, _EMBED), jnp.float32),
        grid=(bs // G,),
        in_specs=[pl.BlockSpec((G, cin, H * W), lambda n: (n, 0, 0)),
                  _const(w0f), _const(b0), _const(w1f), _const(b1),
                  _const(w2f), _const(b2), _const(w3f), _const(b3),
                  _const(p1t), _const(p2t), _const(p3t),
                  _const(wfc_b), _const(bfc2), _const(wqkv), _const(bqkv),
                  _const(wm_b), _const(bm2),
                  _const(g1), _const(be1), _const(g2), _const(be2),
                  _const(wf1_b), _const(bf1_2), _const(wf2_b), _const(bf2_2),
                  _const(c4)],
        out_specs=pl.BlockSpec((G, 1, _EMBED), lambda n: (n, 0, 0)),
        compiler_params=pltpu.CompilerParams(
            dimension_semantics=("parallel",),
            vmem_limit_bytes=64 * 1024 * 1024),
    )(xr, w0f, b0, w1f, b1, w2f, b2, w3f, b3, p1t, p2t, p3t,
      wfc_b, bfc2, wqkv, bqkv, wm_b, bm2,
      g1, be1, g2, be2, wf1_b, bf1_2, wf2_b, bf2_2, c4)

    return out.reshape(bs, _EMBED)


# speedup vs baseline: 1.0745x; 1.0745x over previous
import numpy as np
import jax
import jax.numpy as jnp
from jax import lax
from jax.experimental import pallas as pl
from jax.experimental.pallas import tpu as pltpu

_POOL = 16
_SEQ = 128
_EMBED = 512
_HEADS = 8
_HEAD_DIM = 64
_GROUP = 8


def _fold_bn(gamma, beta, mean, var, eps=1e-5):
    s = gamma / jnp.sqrt(var + eps)
    return s, beta - mean * s


def _adaptive_pool_matrix(n_in, n_out=_POOL):
    m = np.zeros((n_out, n_in), np.float32)
    for i in range(n_out):
        s = (i * n_in) // n_out
        e = -(-((i + 1) * n_in) // n_out)
        m[i, s:e] = 1.0 / (e - s)
    return m


def _pool2d_matrix_t(h_in, w_in):
    return np.kron(_adaptive_pool_matrix(h_in), _adaptive_pool_matrix(w_in)).T.copy()


def _pool_big_matrix_t(gh, gw, stride, row_pitch, n_rows):
    ph_m = _adaptive_pool_matrix(gh)
    pw_m = _adaptive_pool_matrix(gw)
    m = np.zeros((n_rows, _POOL * _POOL), np.float32)
    for ph in range(gh):
        for pw in range(gw):
            q = ph * row_pitch + stride * pw
            m[q, :] = np.kron(ph_m[:, ph], pw_m[:, pw])
    return m


def _gelu_tanh(x):
    c = 0.7978845608028654
    return 0.5 * x * (1.0 + jnp.tanh(c * (x + 0.044715 * x * x * x)))


def _layernorm(x, g, b, eps=1e-5):
    m = jnp.mean(x, axis=-1, keepdims=True)
    c = x - m
    v = jnp.mean(c * c, axis=-1, keepdims=True)
    return c * lax.rsqrt(v + eps) * g + b


def _const(arr):
    zeros = (0,) * arr.ndim
    return pl.BlockSpec(arr.shape, lambda n: zeros)


def _mmT(a, b):
    return lax.dot_general(a, b, (((0,), (0,)), ((), ())),
                           preferred_element_type=jnp.float32)


def _branches(H, W, y, w2_ref, b2_ref, w3_ref, b3_ref, p2_ref, p3_ref):
    C = 32
    h3 = (H - 3) // 3 + 1
    h5 = (H - 5) // 5 + 1
    y3 = y.reshape(H, W, C)

    n2 = h3 * W - 2
    y2g = y3[:3 * h3].reshape(h3, 3, W, C)
    a2 = jnp.concatenate(
        [lax.slice(y2g[:, di].reshape(h3 * W, C), (dj, 0), (dj + n2, C))
         for di in range(3) for dj in range(3)], axis=1)
    z2 = jnp.maximum(b2_ref[...] + jnp.dot(a2, w2_ref[...],
                                           preferred_element_type=jnp.float32),
                     0.0).astype(jnp.bfloat16)
    c2 = _mmT(z2, p2_ref[...])

    n3 = h5 * W - 4
    y3g = y3[:5 * h5].reshape(h5, 5, W, C)
    a3 = jnp.concatenate(
        [lax.slice(y3g[:, di].reshape(h5 * W, C), (dj, 0), (dj + n3, C))
         for di in range(5) for dj in range(5)], axis=1)
    z3 = jnp.maximum(b3_ref[...] + jnp.dot(a3, w3_ref[...],
                                           preferred_element_type=jnp.float32),
                     0.0).astype(jnp.bfloat16)
    c3 = _mmT(z3, p3_ref[...])
    return c2, c3


def _fused_body(H, W, x_ref, w0_ref, b0_ref, w1_ref, b1_ref,
                w2_ref, b2_ref, w3_ref, b3_ref,
                p1_ref, p2_ref, p3_ref,
                wfc_ref, bfc_ref, wqkv_ref, bqkv_ref,
                wm_ref, bm_ref,
                g1_ref, be1_ref, g2_ref, be2_ref,
                wf1_ref, bf1_ref, wf2_ref, bf2_ref, c4_ref, out_ref):
    G = _GROUP
    HW = H * W
    D = _EMBED

    bf = jnp.bfloat16
    ys = [jnp.maximum(_mmT(x_ref[g].astype(bf), w0_ref[...]) + b0_ref[...],
                      0.0).astype(bf)
          for g in range(G)]
    y_all = jnp.concatenate(ys, axis=0)
    z1_all = jnp.maximum(jnp.dot(y_all, w1_ref[...],
                                 preferred_element_type=jnp.float32)
                         + b1_ref[...], 0.0).astype(bf)

    vls = []
    for g in range(G):
        c1 = _mmT(lax.slice(z1_all, (g * HW, 0), ((g + 1) * HW, 32)),
                  p1_ref[...])
        c2, c3 = _branches(H, W, ys[g], w2_ref, b2_ref, w3_ref, b3_ref,
                           p2_ref, p3_ref)
        vls.append(jnp.concatenate([c1, c2, c3], axis=0))
    vl = jnp.concatenate(vls, axis=0).astype(bf)

    xfc = jnp.maximum(jnp.dot(vl, wfc_ref[...],
                              preferred_element_type=jnp.float32) + bfc_ref[...], 0.0)

    xfc_b = xfc.astype(bf)
    qkv = (jnp.dot(xfc_b, wqkv_ref[...],
                   preferred_element_type=jnp.float32) + bqkv_ref[...]).astype(bf)
    scale = 1.0 / (_HEAD_DIM ** 0.5)
    s_list = []
    for g in range(G):
        r0 = g * _SEQ
        for h in range(_HEADS):
            c0 = h * _HEAD_DIM
            kh = lax.slice(qkv, (r0, D + c0), (r0 + _SEQ, D + c0 + _HEAD_DIM))
            vh = lax.slice(qkv, (r0, 2 * D + c0), (r0 + _SEQ, 2 * D + c0 + _HEAD_DIM))
            s_list.append(lax.dot_general(kh, vh, (((1,), (1,)), ((), ())),
                                          preferred_element_type=jnp.float32))
    s_all = jnp.concatenate(s_list, axis=0) * scale
    s_all = s_all - jnp.max(s_all, axis=-1, keepdims=True)
    p_all = jnp.exp(s_all)
    pb = (p_all / jnp.sum(p_all, axis=-1, keepdims=True)).astype(bf)
    o_rows = []
    for g in range(G):
        r0 = g * _SEQ
        p_cat = jnp.concatenate(
            [lax.slice(pb, ((g * _HEADS + h) * _SEQ, 0),
                       ((g * _HEADS + h) * _SEQ + _SEQ, _SEQ))
             for h in range(_HEADS)], axis=1)
        q_bd = jnp.concatenate(
            [jnp.pad(lax.slice(qkv, (r0, h * _HEAD_DIM),
                               (r0 + _SEQ, (h + 1) * _HEAD_DIM)),
                     ((0, 0), (h * _HEAD_DIM, D - (h + 1) * _HEAD_DIM)))
             for h in range(_HEADS)], axis=0)
        o_rows.append(jnp.dot(p_cat, q_bd, preferred_element_type=jnp.float32))
    o = jnp.concatenate(o_rows, axis=0).astype(bf)
    merged = jnp.dot(o, wm_ref[...], preferred_element_type=jnp.float32) + bm_ref[...]

    h1 = _layernorm(xfc + merged, g1_ref[...], be1_ref[...])

    f = _gelu_tanh((jnp.dot(h1.astype(bf), wf1_ref[...],
                            preferred_element_type=jnp.float32)
                    + bf1_ref[...]).astype(bf))
    f = jnp.dot(f, wf2_ref[...],
                preferred_element_type=jnp.float32) + bf2_ref[...]
    h2 = _layernorm(h1 + f, g2_ref[...], be2_ref[...])

    out_ref[...] = jnp.dot(c4_ref[...], h2.astype(bf),
                           preferred_element_type=jnp.float32).reshape(G, 1, D)


def kernel(x, w0, bn0_g, bn0_b, bn0_m, bn0_v, w1, bn1_g, bn1_b, bn1_m, bn1_v,
           w2, bn2_g, bn2_b, bn2_m, bn2_v, w3, bn3_g, bn3_b, bn3_m, bn3_v,
           wfc, bfc, wq, bq, wk, bk, wv, bv, wm, bm,
           ln1_g, ln1_b, ln2_g, ln2_b, wf1, bf1, wf2, bf2, fmp):
    bs, cin, H, W = x.shape
    h3, w3s = (H - 3) // 3 + 1, (W - 3) // 3 + 1
    h5, w5s = (H - 5) // 5 + 1, (W - 5) // 5 + 1
    G = _GROUP
    assert bs % G == 0

    xr = x.reshape(bs, cin, H * W)

    s0, t0 = _fold_bn(bn0_g, bn0_b, bn0_m, bn0_v)
    w0f, b0 = (w0 * s0[None, :]).astype(jnp.bfloat16), t0.reshape(1, -1)
    s1, t1 = _fold_bn(bn1_g, bn1_b, bn1_m, bn1_v)
    w1f, b1 = (w1 * s1[None, :]).astype(jnp.bfloat16), t1.reshape(1, -1)
    s2, t2 = _fold_bn(bn2_g, bn2_b, bn2_m, bn2_v)
    w2f, b2 = (w2 * s2).reshape(9 * 32, 32).astype(jnp.bfloat16), t2.reshape(1, -1)
    s3, t3 = _fold_bn(bn3_g, bn3_b, bn3_m, bn3_v)
    w3f, b3 = (w3 * s3).reshape(25 * 32, 64).astype(jnp.bfloat16), t3.reshape(1, -1)

    p1t = jnp.asarray(_pool2d_matrix_t(H, W)).astype(jnp.bfloat16)
    p2t = jnp.asarray(_pool_big_matrix_t(h3, w3s, 3, W, h3 * W - 2)).astype(jnp.bfloat16)
    p3t = jnp.asarray(_pool_big_matrix_t(h5, w5s, 5, W, h5 * W - 4)).astype(jnp.bfloat16)

    bfc2 = bfc.reshape(1, _EMBED)
    wfc_b = wfc.astype(jnp.bfloat16)
    wm_b = wm.astype(jnp.bfloat16)
    wf1_b = wf1.astype(jnp.bfloat16)
    wf2_b = wf2.astype(jnp.bfloat16)
    wqkv = jnp.concatenate([wq, wk, wv], axis=1).astype(jnp.bfloat16)
    bqkv = jnp.concatenate([bq, bk, bv]).reshape(1, 3 * _EMBED)
    bm2 = bm.reshape(1, _EMBED)
    g1, be1 = ln1_g.reshape(1, _EMBED), ln1_b.reshape(1, _EMBED)
    g2, be2 = ln2_g.reshape(1, _EMBED), ln2_b.reshape(1, _EMBED)
    bf1_2 = bf1.reshape(1, 2 * _EMBED)
    bf2_2 = bf2.reshape(1, _EMBED)
    cvec = jnp.mean(fmp[0], axis=1).reshape(1, _SEQ)
    c4 = jnp.kron(jnp.eye(G, dtype=jnp.float32), cvec).astype(jnp.bfloat16)

    def body(*refs):
        _fused_body(H, W, *refs)

    out = pl.pallas_call(
        body,
        out_shape=jax.ShapeDtypeStruct((bs, 1, _EMBED), jnp.float32),
        grid=(bs // G,),
        in_specs=[pl.BlockSpec((G, cin, H * W), lambda n: (n, 0, 0)),
                  _const(w0f), _const(b0), _const(w1f), _const(b1),
                  _const(w2f), _const(b2), _const(w3f), _const(b3),
                  _const(p1t), _const(p2t), _const(p3t),
                  _const(wfc_b), _const(bfc2), _const(wqkv), _const(bqkv),
                  _const(wm_b), _const(bm2),
                  _const(g1), _const(be1), _const(g2), _const(be2),
                  _const(wf1_b), _const(bf1_2), _const(wf2_b), _const(bf2_2),
                  _const(c4)],
        out_specs=pl.BlockSpec((G, 1, _EMBED), lambda n: (n, 0, 0)),
        compiler_params=pltpu.CompilerParams(
            dimension_semantics=("parallel",),
            vmem_limit_bytes=64 * 1024 * 1024),
    )(xr, w0f, b0, w1f, b1, w2f, b2, w3f, b3, p1t, p2t, p3t,
      wfc_b, bfc2, wqkv, bqkv, wm_b, bm2,
      g1, be1, g2, be2, wf1_b, bf1_2, wf2_b, bf2_2, c4)

    return out.reshape(bs, _EMBED)
